# Initial kernel scaffold; baseline (speedup 1.0000x reference)
import jax
import jax.numpy as jnp
from jax.experimental import pallas as pl


def kernel(x, edge_index, batch, t1_w, t1_b, t2_w, t2_b, att_l_w, att_l_b, att_r_w, att_r_b):
    raise NotImplementedError("write your pallas kernel here")


# speedup vs baseline: 13.6006x; 13.6006x over previous
import functools

import jax
import jax.numpy as jnp
from jax import lax
from jax.experimental import pallas as pl
from jax.experimental.pallas import tpu as pltpu
from jax.experimental.pallas import tpu_sc as plsc

N = 10000
E = 320000
D = 128
H = 128
L = 4
G = 64
EPS = 0.1

NC = 2
NS = 16
LANES = 16
NW = NC * NS
EDGES_PER_TILE = E // NW
CHUNK = 80
NCHUNK = EDGES_PER_TILE // CHUNK
ROWS_PER_TILE = N // NS
ZROWS = 200

_MESH = plsc.VectorSubcoreMesh(core_axis_name="c", subcore_axis_name="s")
_SC_PARAMS = pltpu.CompilerParams(needs_layout_passes=False)


def _tanh(s):
  u = jnp.exp(-2.0 * jnp.abs(s))
  return jnp.sign(s) * (1.0 - u) / (1.0 + u)


@functools.partial(
    pl.kernel,
    out_type=jax.ShapeDtypeStruct((NC * N,), jnp.float32),
    mesh=_MESH,
    compiler_params=_SC_PARAMS,
    scratch_types=[
        pltpu.VMEM((CHUNK,), jnp.int32),
        pltpu.VMEM((CHUNK,), jnp.float32),
        pltpu.VMEM((N,), jnp.float32),
        pltpu.VMEM_SHARED((N,), jnp.float32),
    ],
)
def _sc_degree(dst_hbm, deg_hbm, dst_v, ones_v, zbuf_v, deg_sh):
  cid = lax.axis_index("c")
  sid = lax.axis_index("s")
  wid = cid * NS + sid

  def _zero(i, _):
    zbuf_v[pl.ds(i * LANES, LANES)] = jnp.zeros((LANES,), jnp.float32)
    return 0

  def _ones(i, _):
    ones_v[pl.ds(i * LANES, LANES)] = jnp.ones((LANES,), jnp.float32)
    return 0

  lax.fori_loop(0, CHUNK // LANES, _ones, 0)

  @pl.when(sid == 0)
  def _():
    lax.fori_loop(0, N // LANES, _zero, 0)
    pltpu.sync_copy(zbuf_v, deg_sh)

  plsc.subcore_barrier()

  base = wid * EDGES_PER_TILE

  def _chunk(c, _):
    pltpu.sync_copy(dst_hbm.at[pl.ds(base + c * CHUNK, CHUNK)], dst_v)
    pltpu.sync_copy(ones_v, deg_sh.at[dst_v], add=True)
    return 0

  lax.fori_loop(0, NCHUNK, _chunk, 0)

  plsc.subcore_barrier()

  @pl.when(sid == 0)
  def _():
    pltpu.sync_copy(deg_sh, zbuf_v)
    pltpu.sync_copy(zbuf_v, deg_hbm.at[pl.ds(cid * N, N)])


@functools.partial(
    pl.kernel,
    out_type=jax.ShapeDtypeStruct((NC, N, H), jnp.float32),
    mesh=_MESH,
    compiler_params=_SC_PARAMS,
    scratch_types=[
        pltpu.VMEM((N,), jnp.float32),
        pltpu.VMEM((N,), jnp.float32),
        pltpu.VMEM((N,), jnp.float32),
        pltpu.VMEM((CHUNK,), jnp.int32),
        pltpu.VMEM((CHUNK,), jnp.int32),
        pltpu.VMEM((CHUNK,), jnp.float32),
        pltpu.VMEM((CHUNK, H), jnp.float32),
        pltpu.SemaphoreType.DMA,
        pltpu.VMEM_SHARED((N, H), jnp.float32),
    ],
)
def _sc_layer(h_hbm, al_hbm, ar_hbm, dis_hbm, src_hbm, dst_hbm, agg_hbm,
              al_v, ar_v, dis_v, src_v, dst_v, coef_v, rows_v, sem,
              agg_sh):
  cid = lax.axis_index("c")
  sid = lax.axis_index("s")
  wid = cid * NS + sid

  pltpu.sync_copy(al_hbm, al_v)
  pltpu.sync_copy(ar_hbm, ar_v)
  pltpu.sync_copy(dis_hbm, dis_v)

  def _zero(i, _):
    rows_v[i // (H // LANES), pl.ds((i % (H // LANES)) * LANES, LANES)] = (
        jnp.zeros((LANES,), jnp.float32))
    return 0

  lax.fori_loop(0, CHUNK * (H // LANES), _zero, 0)

  @pl.when(sid == 0)
  def _():
    def _zcopy(k, _):
      pltpu.sync_copy(rows_v, agg_sh.at[pl.ds(k * CHUNK, CHUNK)])
      return 0

    lax.fori_loop(0, N // CHUNK, _zcopy, 0)

  plsc.subcore_barrier()

  base = wid * EDGES_PER_TILE

  def _chunk(c, _):
    off = base + c * CHUNK
    pltpu.sync_copy(src_hbm.at[pl.ds(off, CHUNK)], src_v)
    pltpu.sync_copy(dst_hbm.at[pl.ds(off, CHUNK)], dst_v)
    pltpu.async_copy(h_hbm.at[src_v], rows_v, sem).wait()

    def _coef(g, _):
      s16 = src_v[pl.ds(g * LANES, LANES)]
      d16 = dst_v[pl.ds(g * LANES, LANES)]
      al = plsc.load_gather(al_v, [s16])
      ar = plsc.load_gather(ar_v, [d16])
      t = _tanh(al + ar)
      ew = plsc.load_gather(dis_v, [s16]) * plsc.load_gather(dis_v, [d16])
      coef_v[pl.ds(g * LANES, LANES)] = t * ew
      return 0

    lax.fori_loop(0, CHUNK // LANES, _coef, 0)

    def _scale(e, _):
      b = plsc.load_gather(coef_v, [jnp.full((LANES,), e, jnp.int32)])
      for j in range(H // LANES):
        rows_v[e, pl.ds(j * LANES, LANES)] = (
            rows_v[e, pl.ds(j * LANES, LANES)] * b)
      return 0

    lax.fori_loop(0, CHUNK, _scale, 0)

    pltpu.sync_copy(rows_v, agg_sh.at[dst_v], add=True)
    return 0

  lax.fori_loop(0, NCHUNK, _chunk, 0)

  plsc.subcore_barrier()

  @pl.when(sid == 0)
  def _():
    pltpu.sync_copy(agg_sh, agg_hbm.at[cid])


_BLK = 1000
_NBLK = N // _BLK
_PREC = jax.lax.Precision.HIGHEST


def _tc_prolog_body(x_ref, w1_ref, b1_ref, deg_ref, watt_ref, batt_ref,
                    h_ref, alr_ref, dis_ref):
  h = lax.dot_general(x_ref[...], w1_ref[...], (((1,), (1,)), ((), ())),
                      precision=_PREC) + b1_ref[...]
  h = jnp.maximum(h, 0.0)
  h_ref[...] = h
  alr_ref[...] = lax.dot_general(h, watt_ref[...], (((1,), (0,)), ((), ())),
                                 precision=_PREC) + batt_ref[...]
  deg = deg_ref[...]
  d = deg[:, 0:1] + deg[:, 1:2]
  dis_ref[...] = jnp.where(d > 0.0, lax.rsqrt(jnp.where(d > 0.0, d, 1.0)), 0.0)


def _tc_prolog(x, t1_w, b1, deg2t, watt, batt):
  return pl.pallas_call(
      _tc_prolog_body,
      grid=(_NBLK,),
      in_specs=[
          pl.BlockSpec((_BLK, D), lambda i: (i, 0)),
          pl.BlockSpec((H, D), lambda i: (0, 0)),
          pl.BlockSpec((1, H), lambda i: (0, 0)),
          pl.BlockSpec((_BLK, 2), lambda i: (i, 0)),
          pl.BlockSpec((H, 2), lambda i: (0, 0)),
          pl.BlockSpec((1, 2), lambda i: (0, 0)),
      ],
      out_specs=[
          pl.BlockSpec((_BLK, H), lambda i: (i, 0)),
          pl.BlockSpec((_BLK, 2), lambda i: (i, 0)),
          pl.BlockSpec((_BLK, 1), lambda i: (i, 0)),
      ],
      out_shape=[
          jax.ShapeDtypeStruct((N, H), jnp.float32),
          jax.ShapeDtypeStruct((N, 2), jnp.float32),
          jax.ShapeDtypeStruct((N, 1), jnp.float32),
      ],
  )(x, t1_w, b1, deg2t, watt, batt)


def _tc_combine_body(agg_ref, raw_ref, watt_ref, batt_ref, h_ref, alr_ref):
  h = agg_ref[0] + agg_ref[1] + EPS * raw_ref[...]
  h_ref[...] = h
  alr_ref[...] = lax.dot_general(h, watt_ref[...], (((1,), (0,)), ((), ())),
                                 precision=_PREC) + batt_ref[...]


def _tc_combine(aggp, raw, watt, batt):
  return pl.pallas_call(
      _tc_combine_body,
      grid=(_NBLK,),
      in_specs=[
          pl.BlockSpec((NC, _BLK, H), lambda i: (0, i, 0)),
          pl.BlockSpec((_BLK, H), lambda i: (i, 0)),
          pl.BlockSpec((H, 2), lambda i: (0, 0)),
          pl.BlockSpec((1, 2), lambda i: (0, 0)),
      ],
      out_specs=[
          pl.BlockSpec((_BLK, H), lambda i: (i, 0)),
          pl.BlockSpec((_BLK, 2), lambda i: (i, 0)),
      ],
      out_shape=[
          jax.ShapeDtypeStruct((N, H), jnp.float32),
          jax.ShapeDtypeStruct((N, 2), jnp.float32),
      ],
  )(aggp, raw, watt, batt)


def _tc_epilog_body(agg_ref, raw_ref, w2_ref, b2_ref, batch_ref,
                    h_ref, gemb_ref):
  i = pl.program_id(0)
  h = agg_ref[0] + agg_ref[1] + EPS * raw_ref[...]
  oh = lax.dot_general(h, w2_ref[...], (((1,), (1,)), ((), ())),
                       precision=_PREC) + b2_ref[...]
  h_ref[...] = oh
  gids = lax.broadcasted_iota(jnp.int32, (1, G), 1)
  onehot = (batch_ref[...] == gids).astype(jnp.float32)
  contrib = lax.dot_general(onehot, oh, (((0,), (0,)), ((), ())),
                            precision=_PREC)

  @pl.when(i == 0)
  def _():
    gemb_ref[...] = jnp.zeros_like(gemb_ref)

  gemb_ref[...] += contrib


def _tc_epilog(aggp, raw, t2_w, b2, batch2):
  return pl.pallas_call(
      _tc_epilog_body,
      grid=(_NBLK,),
      in_specs=[
          pl.BlockSpec((NC, _BLK, H), lambda i: (0, i, 0)),
          pl.BlockSpec((_BLK, H), lambda i: (i, 0)),
          pl.BlockSpec((H, H), lambda i: (0, 0)),
          pl.BlockSpec((1, H), lambda i: (0, 0)),
          pl.BlockSpec((_BLK, 1), lambda i: (i, 0)),
      ],
      out_specs=[
          pl.BlockSpec((_BLK, H), lambda i: (i, 0)),
          pl.BlockSpec((G, H), lambda i: (0, 0)),
      ],
      out_shape=[
          jax.ShapeDtypeStruct((N, H), jnp.float32),
          jax.ShapeDtypeStruct((G, H), jnp.float32),
      ],
  )(aggp, raw, t2_w, b2, batch2)


def kernel(x, edge_index, batch, t1_w, t1_b, t2_w, t2_b,
           att_l_w, att_l_b, att_r_w, att_r_b):
  src = edge_index[0]
  dst = edge_index[1]

  watts = [jnp.stack([att_l_w[l], att_r_w[l]], axis=1) for l in range(L)]
  batts = [jnp.stack([att_l_b[l], att_r_b[l]]).reshape(1, 2) for l in range(L)]
  b1 = t1_b.reshape(1, H)
  b2 = t2_b.reshape(1, H)
  batch2 = batch.reshape(N, 1)

  degp = _sc_degree(dst).reshape(NC, N)
  h, alr, dis2 = _tc_prolog(x, t1_w, b1, degp.T, watts[0], batts[0])
  raw = h
  dis = dis2.reshape(N)

  for l in range(L):
    aggp = _sc_layer(h, alr[:, 0], alr[:, 1], dis, src, dst)
    if l < L - 1:
      h, alr = _tc_combine(aggp, raw, watts[l + 1], batts[l + 1])

  out_h, graph_emb = _tc_epilog(aggp, raw, t2_w, b2, batch2)
  return (graph_emb, out_h)
